# baseline (speedup 1.0000x reference)
import dataclasses

import jax
import jax.numpy as jnp
from jax import lax
from jax.experimental import pallas as pl
from jax.experimental.pallas import tpu as pltpu
from jax.experimental.pallas import tpu_sc as plsc

N = 10000
NPAD = 10240
D = 256
HEADS = 8
DPH = 32
E_RAW = 160000
E_TOT = E_RAW + N
NSUB = 16
EPT = 172032
Q = EPT // NSUB
BLK = 128
NB = Q // BLK
RPW = NPAD // NSUB
BR = 256
GRID = NPAD // BR

_f32 = jnp.float32


def _pre_body(x_ref, w_ref, asrc_ref, adst_ref, h0_ref, h1_ref, as_ref, ad_ref):
    i = pl.program_id(0)
    h = jnp.dot(x_ref[...], w_ref[...], preferred_element_type=_f32)
    h0_ref[...] = h[:, :128]
    h1_ref[...] = h[:, 128:]
    k = lax.broadcasted_iota(jnp.int32, (D, 16), 0)
    j = lax.broadcasted_iota(jnp.int32, (D, 16), 1)
    g16 = (k // DPH == j).astype(_f32)
    asv = jnp.dot(h * asrc_ref[...], g16, preferred_element_type=_f32)
    adv = jnp.dot(h * adst_ref[...], g16, preferred_element_type=_f32)
    rid = i * BR + lax.broadcasted_iota(jnp.int32, (BR, 16), 0)
    col = lax.broadcasted_iota(jnp.int32, (BR, 16), 1)
    ok = (rid < N) & (col < HEADS)
    as_ref[...] = jnp.where(ok, asv, -1e30)
    ad_ref[...] = jnp.where(ok, adv, 0.0)


@jax.jit
def _tc_pre(xp, W, a_src, a_dst):
    aflat_s = a_src.reshape(1, D)
    aflat_d = a_dst.reshape(1, D)
    return pl.pallas_call(
        _pre_body,
        grid=(GRID,),
        in_specs=[
            pl.BlockSpec((BR, D), lambda i: (i, 0)),
            pl.BlockSpec((D, D), lambda i: (0, 0)),
            pl.BlockSpec((1, D), lambda i: (0, 0)),
            pl.BlockSpec((1, D), lambda i: (0, 0)),
        ],
        out_specs=[
            pl.BlockSpec((BR, 128), lambda i: (i, 0)),
            pl.BlockSpec((BR, 128), lambda i: (i, 0)),
            pl.BlockSpec((BR, 16), lambda i: (i, 0)),
            pl.BlockSpec((BR, 16), lambda i: (i, 0)),
        ],
        out_shape=[
            jax.ShapeDtypeStruct((NPAD, 128), _f32),
            jax.ShapeDtypeStruct((NPAD, 128), _f32),
            jax.ShapeDtypeStruct((NPAD, 16), _f32),
            jax.ShapeDtypeStruct((NPAD, 16), _f32),
        ],
    )(xp, W, aflat_s, aflat_d)


def _elu(x):
    return jnp.where(x > 0, x, jnp.exp(x) - 1.0)


def _ln(t, g, b):
    mu = jnp.mean(t, axis=-1, keepdims=True)
    var = jnp.mean((t - mu) * (t - mu), axis=-1, keepdims=True)
    return (t - mu) / jnp.sqrt(var + 1e-5) * g + b


def _normalize(a0, a1, den):
    acc = jnp.concatenate([a0, a1], axis=1)
    dinv = 1.0 / (den + 1e-16)
    j = lax.broadcasted_iota(jnp.int32, (16, D), 0)
    cgrp = lax.broadcasted_iota(jnp.int32, (16, D), 1) // DPH
    t16 = (j == cgrp).astype(_f32)
    dexp = jnp.dot(dinv, t16, preferred_element_type=_f32)
    return acc * dexp


def _post1_body(a0_ref, a1_ref, den_ref, b_ref, o_ref):
    out = _normalize(a0_ref[...], a1_ref[...], den_ref[...])
    o_ref[...] = _elu(out + b_ref[...])


@jax.jit
def _tc_post1(acc0, acc1, den, b):
    return pl.pallas_call(
        _post1_body,
        grid=(GRID,),
        in_specs=[
            pl.BlockSpec((BR, 128), lambda i: (i, 0)),
            pl.BlockSpec((BR, 128), lambda i: (i, 0)),
            pl.BlockSpec((BR, 16), lambda i: (i, 0)),
            pl.BlockSpec((1, D), lambda i: (0, 0)),
        ],
        out_specs=pl.BlockSpec((BR, D), lambda i: (i, 0)),
        out_shape=jax.ShapeDtypeStruct((NPAD, D), _f32),
    )(acc0, acc1, den, b.reshape(1, D))


def _mid_body(h_ref, a0_ref, a1_ref, den_ref, bc_ref, g1_ref, b1_ref, w1_ref,
              fb1_ref, w2_ref, fb2_ref, g2_ref, b2_ref, o_ref):
    conv = _normalize(a0_ref[...], a1_ref[...], den_ref[...])
    x1 = _elu(conv + bc_ref[...])
    t = _ln(h_ref[...] + x1, g1_ref[...], b1_ref[...])
    u = jnp.maximum(jnp.dot(t, w1_ref[...], preferred_element_type=_f32)
                    + fb1_ref[...], 0.0)
    u = jnp.dot(u, w2_ref[...], preferred_element_type=_f32) + fb2_ref[...]
    o_ref[...] = _ln(t + u, g2_ref[...], b2_ref[...])


@jax.jit
def _tc_mid(h, acc0, acc1, den, bconv, g1, b1, W1, fb1, W2, fb2, g2, b2):
    vec = lambda v: v.reshape(1, D)
    return pl.pallas_call(
        _mid_body,
        grid=(GRID,),
        in_specs=[
            pl.BlockSpec((BR, D), lambda i: (i, 0)),
            pl.BlockSpec((BR, 128), lambda i: (i, 0)),
            pl.BlockSpec((BR, 128), lambda i: (i, 0)),
            pl.BlockSpec((BR, 16), lambda i: (i, 0)),
            pl.BlockSpec((1, D), lambda i: (0, 0)),
            pl.BlockSpec((1, D), lambda i: (0, 0)),
            pl.BlockSpec((1, D), lambda i: (0, 0)),
            pl.BlockSpec((D, D), lambda i: (0, 0)),
            pl.BlockSpec((1, D), lambda i: (0, 0)),
            pl.BlockSpec((D, D), lambda i: (0, 0)),
            pl.BlockSpec((1, D), lambda i: (0, 0)),
            pl.BlockSpec((1, D), lambda i: (0, 0)),
            pl.BlockSpec((1, D), lambda i: (0, 0)),
        ],
        out_specs=pl.BlockSpec((BR, D), lambda i: (i, 0)),
        out_shape=jax.ShapeDtypeStruct((NPAD, D), _f32),
    )(h, acc0, acc1, den, vec(bconv), vec(g1), vec(b1), W1, vec(fb1), W2,
      vec(fb2), vec(g2), vec(b2))


def _sc_body(as_hbm, ad_hbm, h0_hbm, h1_hbm, src_hbm, dst_hbm,
             acc0_hbm, acc1_hbm, den_hbm,
             idxs_v, idxd_v, asg, adg, eeb, hbuf,
             acc_sp, den_sp, sem0, sem1):
    c = lax.axis_index("c")
    s = lax.axis_index("s")
    zeros16 = jnp.zeros((16,), _f32)

    @pl.loop(0, BLK)
    def _(r):
        eeb[r, :] = zeros16

        @pl.loop(0, 8)
        def _(k):
            hbuf[r, pl.ds(k * 16, 16)] = zeros16

    @pl.loop(0, RPW // BLK)
    def _(j):
        row = s * RPW + j * BLK
        pltpu.sync_copy(hbuf, acc_sp.at[pl.ds(row, BLK)])
        pltpu.sync_copy(eeb, den_sp.at[pl.ds(row, BLK)])

    plsc.subcore_barrier()

    def issue_idx(nb, slot):
        base = s * Q + nb * BLK
        pltpu.async_copy(src_hbm.at[pl.ds(base, BLK)], idxs_v.at[slot], sem1)
        pltpu.async_copy(dst_hbm.at[pl.ds(base, BLK)], idxd_v.at[slot], sem1)

    def wait_idx():
        pltpu.make_async_copy(src_hbm.at[pl.ds(0, BLK)], idxs_v.at[0],
                              sem1).wait()
        pltpu.make_async_copy(dst_hbm.at[pl.ds(0, BLK)], idxd_v.at[0],
                              sem1).wait()

    for half in range(2):
        @pl.when(c == half)
        def _(half=half):
            h_hbm = (h0_hbm, h1_hbm)[half]

            issue_idx(0, 0)

            @pl.loop(0, NB)
            def _(nb):
                par = lax.rem(nb, 2)
                isr = idxs_v.at[par]
                idr = idxd_v.at[par]
                wait_idx()
                c3 = pltpu.async_copy(as_hbm.at[isr], asg, sem0)
                c4 = pltpu.async_copy(ad_hbm.at[idr], adg, sem0)
                c5 = pltpu.async_copy(h_hbm.at[isr], hbuf, sem0)

                @pl.when(nb < NB - 1)
                def _():
                    issue_idx(nb + 1, 1 - par)

                c3.wait()
                c4.wait()

                @pl.loop(0, BLK)
                def _(r):
                    e = asg[r, :] + adg[r, :]
                    e = jnp.maximum(e, e * 0.2)
                    eeb[r, :] = jnp.exp(e)

                c5.wait()

                @pl.loop(0, BLK, step=2)
                def _(r0):
                    for dr in range(2):
                        r = r0 + dr
                        ridx = jnp.broadcast_to(r, (16,)).astype(jnp.int32)
                        for hk in range(4):
                            head = half * 4 + hk
                            hidx = jnp.full((16,), head, jnp.int32)
                            al = plsc.load_gather(eeb, [ridx, hidx])
                            c0 = hk * 32
                            hbuf[r, pl.ds(c0, 16)] = (
                                hbuf[r, pl.ds(c0, 16)] * al)
                            hbuf[r, pl.ds(c0 + 16, 16)] = (
                                hbuf[r, pl.ds(c0 + 16, 16)] * al)

                pltpu.sync_copy(hbuf, acc_sp.at[idr], add=True)

                @pl.when(c == 0)
                def _():
                    pltpu.sync_copy(eeb, den_sp.at[idr], add=True)

    plsc.subcore_barrier()

    for half in range(2):
        @pl.when(c == half)
        def _(half=half):
            acc_hbm = (acc0_hbm, acc1_hbm)[half]

            @pl.loop(0, RPW // BLK)
            def _(j):
                row = s * RPW + j * BLK
                pltpu.sync_copy(acc_sp.at[pl.ds(row, BLK)],
                                acc_hbm.at[pl.ds(row, BLK)])

    @pl.when(c == 0)
    def _():
        @pl.loop(0, RPW // BLK)
        def _(j):
            row = s * RPW + j * BLK
            pltpu.sync_copy(den_sp.at[pl.ds(row, BLK)],
                            den_hbm.at[pl.ds(row, BLK)])


def _make_sc_sparse():
    cp = pltpu.CompilerParams()
    fields = pltpu.CompilerParams.__dataclass_fields__
    if "needs_layout_passes" in fields:
        cp = dataclasses.replace(cp, needs_layout_passes=False)
    if "use_tc_tiling_on_sc" in fields:
        cp = dataclasses.replace(cp, use_tc_tiling_on_sc=False)
    mesh = plsc.VectorSubcoreMesh(core_axis_name="c", subcore_axis_name="s")
    return pl.kernel(
        _sc_body,
        out_type=(
            jax.ShapeDtypeStruct((NPAD, 128), _f32),
            jax.ShapeDtypeStruct((NPAD, 128), _f32),
            jax.ShapeDtypeStruct((NPAD, 16), _f32),
        ),
        mesh=mesh,
        scratch_types=[
            pltpu.VMEM((2, BLK), jnp.int32),
            pltpu.VMEM((2, BLK), jnp.int32),
            pltpu.VMEM((BLK, 16), _f32),
            pltpu.VMEM((BLK, 16), _f32),
            pltpu.VMEM((BLK, 16), _f32),
            pltpu.VMEM((BLK, 128), _f32),
            pltpu.VMEM_SHARED((NPAD, 128), _f32),
            pltpu.VMEM_SHARED((NPAD, 16), _f32),
            pltpu.SemaphoreType.DMA,
            pltpu.SemaphoreType.DMA,
        ],
        compiler_params=cp,
    )


_sc_sparse = _make_sc_sparse()


def kernel(x, edge_index, params):
    src = edge_index[0].astype(jnp.int32)
    dst = edge_index[1].astype(jnp.int32)
    loops = jnp.arange(N, dtype=jnp.int32)
    srcp = jnp.full((EPT,), NPAD - 1, jnp.int32).at[:E_TOT].set(
        jnp.concatenate([src, loops]))
    dstp = jnp.full((EPT,), NPAD - 1, jnp.int32).at[:E_TOT].set(
        jnp.concatenate([dst, loops]))
    xp = jnp.zeros((NPAD, D), _f32).at[:N].set(x)

    p1 = params['conv1']
    h0, h1, as16, ad16 = _tc_pre(xp, p1['W'], p1['a_src'], p1['a_dst'])
    acc0, acc1, den = _sc_sparse(as16, ad16, h0, h1, srcp, dstp)
    h = _tc_post1(acc0, acc1, den, p1['b'])

    for i in range(5):
        pc = params['convs'][i]
        h0, h1, as16, ad16 = _tc_pre(h, pc['W'], pc['a_src'], pc['a_dst'])
        acc0, acc1, den = _sc_sparse(as16, ad16, h0, h1, srcp, dstp)
        fc = params['fcs'][i]
        nm = params['norms'][i]
        fn = params['fc_norms'][i]
        h = _tc_mid(h, acc0, acc1, den, pc['b'], nm['g'], nm['b'],
                    fc['W1'], fc['b1'], fc['W2'], fc['b2'],
                    fn['g'], fn['b'])

    return h[:N]


# speedup vs baseline: 1.0081x; 1.0081x over previous
import dataclasses

import jax
import jax.numpy as jnp
from jax import lax
from jax.experimental import pallas as pl
from jax.experimental.pallas import tpu as pltpu
from jax.experimental.pallas import tpu_sc as plsc

N = 10000
NPAD = 10240
D = 256
HEADS = 8
DPH = 32
E_RAW = 160000
E_TOT = E_RAW + N
NSUB = 16
EPT = 172032
Q = EPT // NSUB
BLK = 128
NB = Q // BLK
RPW = NPAD // NSUB
BR = 256
GRID = NPAD // BR

_f32 = jnp.float32


def _pre_body(x_ref, w_ref, asrc_ref, adst_ref, h0_ref, h1_ref, as_ref, ad_ref):
    i = pl.program_id(0)
    h = jnp.dot(x_ref[...], w_ref[...], preferred_element_type=_f32)
    h0_ref[...] = h[:, :128]
    h1_ref[...] = h[:, 128:]
    k = lax.broadcasted_iota(jnp.int32, (D, 16), 0)
    j = lax.broadcasted_iota(jnp.int32, (D, 16), 1)
    g16 = (k // DPH == j).astype(_f32)
    asv = jnp.dot(h * asrc_ref[...], g16, preferred_element_type=_f32)
    adv = jnp.dot(h * adst_ref[...], g16, preferred_element_type=_f32)
    rid = i * BR + lax.broadcasted_iota(jnp.int32, (BR, 16), 0)
    col = lax.broadcasted_iota(jnp.int32, (BR, 16), 1)
    ok = (rid < N) & (col < HEADS)
    as_ref[...] = jnp.where(ok, asv, -1e30)
    ad_ref[...] = jnp.where(ok, adv, 0.0)


@jax.jit
def _tc_pre(xp, W, a_src, a_dst):
    aflat_s = a_src.reshape(1, D)
    aflat_d = a_dst.reshape(1, D)
    return pl.pallas_call(
        _pre_body,
        grid=(GRID,),
        in_specs=[
            pl.BlockSpec((BR, D), lambda i: (i, 0)),
            pl.BlockSpec((D, D), lambda i: (0, 0)),
            pl.BlockSpec((1, D), lambda i: (0, 0)),
            pl.BlockSpec((1, D), lambda i: (0, 0)),
        ],
        out_specs=[
            pl.BlockSpec((BR, 128), lambda i: (i, 0)),
            pl.BlockSpec((BR, 128), lambda i: (i, 0)),
            pl.BlockSpec((BR, 16), lambda i: (i, 0)),
            pl.BlockSpec((BR, 16), lambda i: (i, 0)),
        ],
        out_shape=[
            jax.ShapeDtypeStruct((NPAD, 128), _f32),
            jax.ShapeDtypeStruct((NPAD, 128), _f32),
            jax.ShapeDtypeStruct((NPAD, 16), _f32),
            jax.ShapeDtypeStruct((NPAD, 16), _f32),
        ],
    )(xp, W, aflat_s, aflat_d)


def _elu(x):
    return jnp.where(x > 0, x, jnp.exp(x) - 1.0)


def _ln(t, g, b):
    mu = jnp.mean(t, axis=-1, keepdims=True)
    var = jnp.mean((t - mu) * (t - mu), axis=-1, keepdims=True)
    return (t - mu) / jnp.sqrt(var + 1e-5) * g + b


def _normalize(a0, a1, den):
    acc = jnp.concatenate([a0, a1], axis=1)
    dinv = 1.0 / (den + 1e-16)
    j = lax.broadcasted_iota(jnp.int32, (16, D), 0)
    cgrp = lax.broadcasted_iota(jnp.int32, (16, D), 1) // DPH
    t16 = (j == cgrp).astype(_f32)
    dexp = jnp.dot(dinv, t16, preferred_element_type=_f32)
    return acc * dexp


def _post1_body(a0_ref, a1_ref, den_ref, b_ref, o_ref):
    out = _normalize(a0_ref[...], a1_ref[...], den_ref[...])
    o_ref[...] = _elu(out + b_ref[...])


@jax.jit
def _tc_post1(acc0, acc1, den, b):
    return pl.pallas_call(
        _post1_body,
        grid=(GRID,),
        in_specs=[
            pl.BlockSpec((BR, 128), lambda i: (i, 0)),
            pl.BlockSpec((BR, 128), lambda i: (i, 0)),
            pl.BlockSpec((BR, 16), lambda i: (i, 0)),
            pl.BlockSpec((1, D), lambda i: (0, 0)),
        ],
        out_specs=pl.BlockSpec((BR, D), lambda i: (i, 0)),
        out_shape=jax.ShapeDtypeStruct((NPAD, D), _f32),
    )(acc0, acc1, den, b.reshape(1, D))


def _mid_body(h_ref, a0_ref, a1_ref, den_ref, bc_ref, g1_ref, b1_ref, w1_ref,
              fb1_ref, w2_ref, fb2_ref, g2_ref, b2_ref, o_ref):
    conv = _normalize(a0_ref[...], a1_ref[...], den_ref[...])
    x1 = _elu(conv + bc_ref[...])
    t = _ln(h_ref[...] + x1, g1_ref[...], b1_ref[...])
    u = jnp.maximum(jnp.dot(t, w1_ref[...], preferred_element_type=_f32)
                    + fb1_ref[...], 0.0)
    u = jnp.dot(u, w2_ref[...], preferred_element_type=_f32) + fb2_ref[...]
    o_ref[...] = _ln(t + u, g2_ref[...], b2_ref[...])


@jax.jit
def _tc_mid(h, acc0, acc1, den, bconv, g1, b1, W1, fb1, W2, fb2, g2, b2):
    vec = lambda v: v.reshape(1, D)
    return pl.pallas_call(
        _mid_body,
        grid=(GRID,),
        in_specs=[
            pl.BlockSpec((BR, D), lambda i: (i, 0)),
            pl.BlockSpec((BR, 128), lambda i: (i, 0)),
            pl.BlockSpec((BR, 128), lambda i: (i, 0)),
            pl.BlockSpec((BR, 16), lambda i: (i, 0)),
            pl.BlockSpec((1, D), lambda i: (0, 0)),
            pl.BlockSpec((1, D), lambda i: (0, 0)),
            pl.BlockSpec((1, D), lambda i: (0, 0)),
            pl.BlockSpec((D, D), lambda i: (0, 0)),
            pl.BlockSpec((1, D), lambda i: (0, 0)),
            pl.BlockSpec((D, D), lambda i: (0, 0)),
            pl.BlockSpec((1, D), lambda i: (0, 0)),
            pl.BlockSpec((1, D), lambda i: (0, 0)),
            pl.BlockSpec((1, D), lambda i: (0, 0)),
        ],
        out_specs=pl.BlockSpec((BR, D), lambda i: (i, 0)),
        out_shape=jax.ShapeDtypeStruct((NPAD, D), _f32),
    )(h, acc0, acc1, den, vec(bconv), vec(g1), vec(b1), W1, vec(fb1), W2,
      vec(fb2), vec(g2), vec(b2))


def _sc_body(as_hbm, ad_hbm, h0_hbm, h1_hbm, src_hbm, dst_hbm,
             acc0_hbm, acc1_hbm, den_hbm,
             idxs_v, idxd_v, asg, adg, eeb, hbuf,
             acc_sp, den_sp, sem0, sem1):
    c = lax.axis_index("c")
    s = lax.axis_index("s")
    zeros16 = jnp.zeros((16,), _f32)

    @pl.loop(0, BLK)
    def _(r):
        eeb[r, :] = zeros16

        @pl.loop(0, 8)
        def _(k):
            hbuf[r, pl.ds(k * 16, 16)] = zeros16

    @pl.loop(0, RPW // BLK)
    def _(j):
        row = s * RPW + j * BLK
        pltpu.sync_copy(hbuf, acc_sp.at[pl.ds(row, BLK)])
        pltpu.sync_copy(eeb, den_sp.at[pl.ds(row, BLK)])

    plsc.subcore_barrier()

    def issue_idx(nb, slot):
        base = s * Q + nb * BLK
        pltpu.async_copy(src_hbm.at[pl.ds(base, BLK)], idxs_v.at[slot], sem1)
        pltpu.async_copy(dst_hbm.at[pl.ds(base, BLK)], idxd_v.at[slot], sem1)

    def wait_idx():
        pltpu.make_async_copy(src_hbm.at[pl.ds(0, BLK)], idxs_v.at[0],
                              sem1).wait()
        pltpu.make_async_copy(dst_hbm.at[pl.ds(0, BLK)], idxd_v.at[0],
                              sem1).wait()

    for half in range(2):
        @pl.when(c == half)
        def _(half=half):
            h_hbm = (h0_hbm, h1_hbm)[half]

            issue_idx(0, 0)

            @pl.loop(0, NB)
            def _(nb):
                par = lax.rem(nb, 2)
                isr = idxs_v.at[par]
                idr = idxd_v.at[par]
                wait_idx()
                c3 = pltpu.async_copy(as_hbm.at[isr], asg, sem0)
                c4 = pltpu.async_copy(ad_hbm.at[idr], adg, sem0)
                c5 = pltpu.async_copy(h_hbm.at[isr], hbuf, sem0)

                @pl.when(nb < NB - 1)
                def _():
                    issue_idx(nb + 1, 1 - par)

                c3.wait()
                c4.wait()

                @pl.loop(0, BLK)
                def _(r):
                    e = asg[r, :] + adg[r, :]
                    e = jnp.maximum(e, e * 0.2)
                    eeb[r, :] = jnp.exp(e)

                c5.wait()

                @pl.loop(0, BLK)
                def _(r):
                    ridx = jnp.broadcast_to(r, (16,)).astype(jnp.int32)
                    for hk in range(4):
                        head = half * 4 + hk
                        hidx = jnp.full((16,), head, jnp.int32)
                        al = plsc.load_gather(eeb, [ridx, hidx])
                        c0 = hk * 32
                        hbuf[r, pl.ds(c0, 16)] = hbuf[r, pl.ds(c0, 16)] * al
                        hbuf[r, pl.ds(c0 + 16, 16)] = (
                            hbuf[r, pl.ds(c0 + 16, 16)] * al)

                pltpu.sync_copy(hbuf, acc_sp.at[idr], add=True)

                @pl.when(c == 0)
                def _():
                    pltpu.sync_copy(eeb, den_sp.at[idr], add=True)

    plsc.subcore_barrier()

    for half in range(2):
        @pl.when(c == half)
        def _(half=half):
            acc_hbm = (acc0_hbm, acc1_hbm)[half]

            @pl.loop(0, RPW // BLK)
            def _(j):
                row = s * RPW + j * BLK
                pltpu.sync_copy(acc_sp.at[pl.ds(row, BLK)],
                                acc_hbm.at[pl.ds(row, BLK)])

    @pl.when(c == 0)
    def _():
        @pl.loop(0, RPW // BLK)
        def _(j):
            row = s * RPW + j * BLK
            pltpu.sync_copy(den_sp.at[pl.ds(row, BLK)],
                            den_hbm.at[pl.ds(row, BLK)])


def _make_sc_sparse():
    cp = pltpu.CompilerParams()
    fields = pltpu.CompilerParams.__dataclass_fields__
    if "needs_layout_passes" in fields:
        cp = dataclasses.replace(cp, needs_layout_passes=False)
    if "use_tc_tiling_on_sc" in fields:
        cp = dataclasses.replace(cp, use_tc_tiling_on_sc=False)
    mesh = plsc.VectorSubcoreMesh(core_axis_name="c", subcore_axis_name="s")
    return pl.kernel(
        _sc_body,
        out_type=(
            jax.ShapeDtypeStruct((NPAD, 128), _f32),
            jax.ShapeDtypeStruct((NPAD, 128), _f32),
            jax.ShapeDtypeStruct((NPAD, 16), _f32),
        ),
        mesh=mesh,
        scratch_types=[
            pltpu.VMEM((2, BLK), jnp.int32),
            pltpu.VMEM((2, BLK), jnp.int32),
            pltpu.VMEM((BLK, 16), _f32),
            pltpu.VMEM((BLK, 16), _f32),
            pltpu.VMEM((BLK, 16), _f32),
            pltpu.VMEM((BLK, 128), _f32),
            pltpu.VMEM_SHARED((NPAD, 128), _f32),
            pltpu.VMEM_SHARED((NPAD, 16), _f32),
            pltpu.SemaphoreType.DMA,
            pltpu.SemaphoreType.DMA,
        ],
        compiler_params=cp,
    )


_sc_sparse = _make_sc_sparse()


def kernel(x, edge_index, params):
    src = edge_index[0].astype(jnp.int32)
    dst = edge_index[1].astype(jnp.int32)
    loops = jnp.arange(N, dtype=jnp.int32)
    srcp = jnp.full((EPT,), NPAD - 1, jnp.int32).at[:E_TOT].set(
        jnp.concatenate([src, loops]))
    dstp = jnp.full((EPT,), NPAD - 1, jnp.int32).at[:E_TOT].set(
        jnp.concatenate([dst, loops]))
    xp = jnp.zeros((NPAD, D), _f32).at[:N].set(x)

    p1 = params['conv1']
    h0, h1, as16, ad16 = _tc_pre(xp, p1['W'], p1['a_src'], p1['a_dst'])
    acc0, acc1, den = _sc_sparse(as16, ad16, h0, h1, srcp, dstp)
    h = _tc_post1(acc0, acc1, den, p1['b'])

    for i in range(5):
        pc = params['convs'][i]
        h0, h1, as16, ad16 = _tc_pre(h, pc['W'], pc['a_src'], pc['a_dst'])
        acc0, acc1, den = _sc_sparse(as16, ad16, h0, h1, srcp, dstp)
        fc = params['fcs'][i]
        nm = params['norms'][i]
        fn = params['fc_norms'][i]
        h = _tc_mid(h, acc0, acc1, den, pc['b'], nm['g'], nm['b'],
                    fc['W1'], fc['b1'], fc['W2'], fc['b2'],
                    fn['g'], fn['b'])

    return h[:N]


# speedup vs baseline: 1.0348x; 1.0265x over previous
import dataclasses

import jax
import jax.numpy as jnp
from jax import lax
from jax.experimental import pallas as pl
from jax.experimental.pallas import tpu as pltpu
from jax.experimental.pallas import tpu_sc as plsc

N = 10000
NPAD = 10240
D = 256
HEADS = 8
DPH = 32
E_RAW = 160000
E_TOT = E_RAW + N
NSUB = 16
EPT = 172032
Q = EPT // NSUB
BLK = 128
NB = Q // BLK
RPW = NPAD // NSUB
NSAFE = 10112
NBLK_SAFE = NSAFE // BLK
BR = 256
GRID = NPAD // BR

_f32 = jnp.float32


def _pre_body(x_ref, w_ref, asrc_ref, adst_ref, h0_ref, h1_ref, as_ref, ad_ref):
    i = pl.program_id(0)
    h = jnp.dot(x_ref[...], w_ref[...], preferred_element_type=_f32)
    h0_ref[...] = h[:, :128]
    h1_ref[...] = h[:, 128:]
    k = lax.broadcasted_iota(jnp.int32, (D, 16), 0)
    j = lax.broadcasted_iota(jnp.int32, (D, 16), 1)
    g16 = (k // DPH == j).astype(_f32)
    asv = jnp.dot(h * asrc_ref[...], g16, preferred_element_type=_f32)
    adv = jnp.dot(h * adst_ref[...], g16, preferred_element_type=_f32)
    rid = i * BR + lax.broadcasted_iota(jnp.int32, (BR, 16), 0)
    col = lax.broadcasted_iota(jnp.int32, (BR, 16), 1)
    ok = (rid < N) & (col < HEADS)
    as_ref[...] = jnp.where(ok, asv, -1e30)
    ad_ref[...] = jnp.where(ok, adv, 0.0)


@jax.jit
def _tc_pre(xp, W, a_src, a_dst):
    aflat_s = a_src.reshape(1, D)
    aflat_d = a_dst.reshape(1, D)
    return pl.pallas_call(
        _pre_body,
        grid=(GRID,),
        in_specs=[
            pl.BlockSpec((BR, D), lambda i: (i, 0)),
            pl.BlockSpec((D, D), lambda i: (0, 0)),
            pl.BlockSpec((1, D), lambda i: (0, 0)),
            pl.BlockSpec((1, D), lambda i: (0, 0)),
        ],
        out_specs=[
            pl.BlockSpec((BR, 128), lambda i: (i, 0)),
            pl.BlockSpec((BR, 128), lambda i: (i, 0)),
            pl.BlockSpec((BR, 16), lambda i: (i, 0)),
            pl.BlockSpec((BR, 16), lambda i: (i, 0)),
        ],
        out_shape=[
            jax.ShapeDtypeStruct((NPAD, 128), _f32),
            jax.ShapeDtypeStruct((NPAD, 128), _f32),
            jax.ShapeDtypeStruct((NPAD, 16), _f32),
            jax.ShapeDtypeStruct((NPAD, 16), _f32),
        ],
    )(xp, W, aflat_s, aflat_d)


def _elu(x):
    return jnp.where(x > 0, x, jnp.exp(x) - 1.0)


def _ln(t, g, b):
    mu = jnp.mean(t, axis=-1, keepdims=True)
    var = jnp.mean((t - mu) * (t - mu), axis=-1, keepdims=True)
    return (t - mu) / jnp.sqrt(var + 1e-5) * g + b


def _normalize(a0, a1, den):
    acc = jnp.concatenate([a0, a1], axis=1)
    dinv = 1.0 / (den + 1e-16)
    j = lax.broadcasted_iota(jnp.int32, (16, D), 0)
    cgrp = lax.broadcasted_iota(jnp.int32, (16, D), 1) // DPH
    t16 = (j == cgrp).astype(_f32)
    dexp = jnp.dot(dinv, t16, preferred_element_type=_f32)
    return acc * dexp


def _post1_body(a0_ref, a1_ref, den_ref, b_ref, o_ref):
    out = _normalize(a0_ref[...], a1_ref[...], den_ref[...])
    o_ref[...] = _elu(out + b_ref[...])


@jax.jit
def _tc_post1(acc0, acc1, den, b):
    return pl.pallas_call(
        _post1_body,
        grid=(GRID,),
        in_specs=[
            pl.BlockSpec((BR, 128), lambda i: (i, 0)),
            pl.BlockSpec((BR, 128), lambda i: (i, 0)),
            pl.BlockSpec((BR, 16), lambda i: (i, 0)),
            pl.BlockSpec((1, D), lambda i: (0, 0)),
        ],
        out_specs=pl.BlockSpec((BR, D), lambda i: (i, 0)),
        out_shape=jax.ShapeDtypeStruct((NPAD, D), _f32),
    )(acc0, acc1, den, b.reshape(1, D))


def _mid_body(h_ref, a0_ref, a1_ref, den_ref, bc_ref, g1_ref, b1_ref, w1_ref,
              fb1_ref, w2_ref, fb2_ref, g2_ref, b2_ref, o_ref):
    conv = _normalize(a0_ref[...], a1_ref[...], den_ref[...])
    x1 = _elu(conv + bc_ref[...])
    t = _ln(h_ref[...] + x1, g1_ref[...], b1_ref[...])
    u = jnp.maximum(jnp.dot(t, w1_ref[...], preferred_element_type=_f32)
                    + fb1_ref[...], 0.0)
    u = jnp.dot(u, w2_ref[...], preferred_element_type=_f32) + fb2_ref[...]
    o_ref[...] = _ln(t + u, g2_ref[...], b2_ref[...])


@jax.jit
def _tc_mid(h, acc0, acc1, den, bconv, g1, b1, W1, fb1, W2, fb2, g2, b2):
    vec = lambda v: v.reshape(1, D)
    return pl.pallas_call(
        _mid_body,
        grid=(GRID,),
        in_specs=[
            pl.BlockSpec((BR, D), lambda i: (i, 0)),
            pl.BlockSpec((BR, 128), lambda i: (i, 0)),
            pl.BlockSpec((BR, 128), lambda i: (i, 0)),
            pl.BlockSpec((BR, 16), lambda i: (i, 0)),
            pl.BlockSpec((1, D), lambda i: (0, 0)),
            pl.BlockSpec((1, D), lambda i: (0, 0)),
            pl.BlockSpec((1, D), lambda i: (0, 0)),
            pl.BlockSpec((D, D), lambda i: (0, 0)),
            pl.BlockSpec((1, D), lambda i: (0, 0)),
            pl.BlockSpec((D, D), lambda i: (0, 0)),
            pl.BlockSpec((1, D), lambda i: (0, 0)),
            pl.BlockSpec((1, D), lambda i: (0, 0)),
            pl.BlockSpec((1, D), lambda i: (0, 0)),
        ],
        out_specs=pl.BlockSpec((BR, D), lambda i: (i, 0)),
        out_shape=jax.ShapeDtypeStruct((NPAD, D), _f32),
    )(h, acc0, acc1, den, vec(bconv), vec(g1), vec(b1), W1, vec(fb1), W2,
      vec(fb2), vec(g2), vec(b2))


def _sc_body(as_hbm, ad_hbm, h0_hbm, h1_hbm, src_hbm, dst_hbm,
             acc0_hbm, acc1_hbm, den_hbm,
             idxs_v, idxd_v, asg, adg, eeb, hbuf,
             acc_sp, den_sp, sem0, sem1, sem2):
    c = lax.axis_index("c")
    s = lax.axis_index("s")
    zeros16 = jnp.zeros((16,), _f32)

    @pl.loop(0, BLK)
    def _(r):
        eeb[r, :] = zeros16

        @pl.loop(0, 8)
        def _(k):
            hbuf[0, r, pl.ds(k * 16, 16)] = zeros16

    @pl.loop(0, 5)
    def _(j):
        bi = s * 5 + j

        @pl.when(bi < NBLK_SAFE)
        def _():
            row = bi * BLK
            pltpu.sync_copy(hbuf.at[0], acc_sp.at[pl.ds(row, BLK)])
            pltpu.sync_copy(eeb, den_sp.at[pl.ds(row, BLK)])

    @pl.when(s == 0)
    def _():
        for half in range(2):
            @pl.when(c == half)
            def _(half=half):
                acc_hbm = (acc0_hbm, acc1_hbm)[half]
                pltpu.sync_copy(hbuf.at[0], acc_hbm.at[pl.ds(NSAFE, BLK)])

        @pl.when(c == 0)
        def _():
            pltpu.sync_copy(eeb, den_hbm.at[pl.ds(NSAFE, BLK)])

    plsc.subcore_barrier()

    def issue_idx(nb, slot):
        base = s * Q + nb * BLK
        pltpu.async_copy(src_hbm.at[pl.ds(base, BLK)], idxs_v.at[slot], sem1)
        pltpu.async_copy(dst_hbm.at[pl.ds(base, BLK)], idxd_v.at[slot], sem1)

    def wait_idx():
        pltpu.make_async_copy(src_hbm.at[pl.ds(0, BLK)], idxs_v.at[0],
                              sem1).wait()
        pltpu.make_async_copy(dst_hbm.at[pl.ds(0, BLK)], idxd_v.at[0],
                              sem1).wait()

    for half in range(2):
        @pl.when(c == half)
        def _(half=half):
            h_hbm = (h0_hbm, h1_hbm)[half]

            issue_idx(0, 0)

            @pl.loop(0, NB + 1)
            def _(nb):
                par = lax.rem(nb, 2)
                opar = 1 - par

                @pl.when(nb < NB)
                def _():
                    wait_idx()
                    pltpu.async_copy(h_hbm.at[idxs_v.at[par]],
                                     hbuf.at[par], sem2)

                @pl.when(nb > 0)
                def _():
                    c3 = pltpu.async_copy(as_hbm.at[idxs_v.at[opar]],
                                          asg, sem0)
                    c4 = pltpu.async_copy(ad_hbm.at[idxd_v.at[opar]],
                                          adg, sem0)
                    c3.wait()
                    c4.wait()

                    @pl.loop(0, BLK)
                    def _(r):
                        e = asg[r, :] + adg[r, :]
                        e = jnp.maximum(e, e * 0.2)
                        eeb[r, :] = jnp.exp(e)

                    pltpu.make_async_copy(h_hbm.at[pl.ds(0, BLK)],
                                          hbuf.at[opar], sem2).wait()

                    @pl.loop(0, BLK)
                    def _(r):
                        ridx = jnp.broadcast_to(r, (16,)).astype(jnp.int32)
                        opv = jnp.broadcast_to(opar, (16,)).astype(jnp.int32)
                        for hk in range(4):
                            head = half * 4 + hk
                            hidx = jnp.full((16,), head, jnp.int32)
                            al = plsc.load_gather(eeb, [ridx, hidx])
                            c0 = hk * 32
                            hbuf[opar, r, pl.ds(c0, 16)] = (
                                hbuf[opar, r, pl.ds(c0, 16)] * al)
                            hbuf[opar, r, pl.ds(c0 + 16, 16)] = (
                                hbuf[opar, r, pl.ds(c0 + 16, 16)] * al)

                    pltpu.sync_copy(hbuf.at[opar],
                                    acc_sp.at[idxd_v.at[opar]], add=True)

                    @pl.when(c == 0)
                    def _():
                        pltpu.sync_copy(eeb, den_sp.at[idxd_v.at[opar]],
                                        add=True)

                @pl.when(nb < NB - 1)
                def _():
                    issue_idx(nb + 1, opar)

    plsc.subcore_barrier()

    for half in range(2):
        @pl.when(c == half)
        def _(half=half):
            acc_hbm = (acc0_hbm, acc1_hbm)[half]

            @pl.loop(0, 5)
            def _(j):
                bi = s * 5 + j

                @pl.when(bi < NBLK_SAFE)
                def _():
                    row = bi * BLK
                    pltpu.sync_copy(acc_sp.at[pl.ds(row, BLK)],
                                    acc_hbm.at[pl.ds(row, BLK)])

    @pl.when(c == 0)
    def _():
        @pl.loop(0, 5)
        def _(j):
            bi = s * 5 + j

            @pl.when(bi < NBLK_SAFE)
            def _():
                row = bi * BLK
                pltpu.sync_copy(den_sp.at[pl.ds(row, BLK)],
                                den_hbm.at[pl.ds(row, BLK)])


def _make_sc_sparse():
    cp = pltpu.CompilerParams()
    fields = pltpu.CompilerParams.__dataclass_fields__
    if "needs_layout_passes" in fields:
        cp = dataclasses.replace(cp, needs_layout_passes=False)
    if "use_tc_tiling_on_sc" in fields:
        cp = dataclasses.replace(cp, use_tc_tiling_on_sc=False)
    if "internal_scratch_in_bytes" in fields:
        cp = dataclasses.replace(cp, internal_scratch_in_bytes=0)
    mesh = plsc.VectorSubcoreMesh(core_axis_name="c", subcore_axis_name="s")
    return pl.kernel(
        _sc_body,
        out_type=(
            jax.ShapeDtypeStruct((NPAD, 128), _f32),
            jax.ShapeDtypeStruct((NPAD, 128), _f32),
            jax.ShapeDtypeStruct((NPAD, 16), _f32),
        ),
        mesh=mesh,
        scratch_types=[
            pltpu.VMEM((2, BLK), jnp.int32),
            pltpu.VMEM((2, BLK), jnp.int32),
            pltpu.VMEM((BLK, 16), _f32),
            pltpu.VMEM((BLK, 16), _f32),
            pltpu.VMEM((BLK, 16), _f32),
            pltpu.VMEM((2, BLK, 128), _f32),
            pltpu.VMEM_SHARED((NSAFE, 128), _f32),
            pltpu.VMEM_SHARED((NSAFE, 16), _f32),
            pltpu.SemaphoreType.DMA,
            pltpu.SemaphoreType.DMA,
            pltpu.SemaphoreType.DMA,
        ],
        compiler_params=cp,
    )


_sc_sparse = _make_sc_sparse()


def kernel(x, edge_index, params):
    src = edge_index[0].astype(jnp.int32)
    dst = edge_index[1].astype(jnp.int32)
    loops = jnp.arange(N, dtype=jnp.int32)
    srcp = jnp.full((EPT,), NPAD - 1, jnp.int32).at[:E_TOT].set(
        jnp.concatenate([src, loops]))
    dstp = jnp.zeros((EPT,), jnp.int32).at[:E_TOT].set(
        jnp.concatenate([dst, loops]))
    xp = jnp.zeros((NPAD, D), _f32).at[:N].set(x)

    p1 = params['conv1']
    h0, h1, as16, ad16 = _tc_pre(xp, p1['W'], p1['a_src'], p1['a_dst'])
    acc0, acc1, den = _sc_sparse(as16, ad16, h0, h1, srcp, dstp)
    h = _tc_post1(acc0, acc1, den, p1['b'])

    for i in range(5):
        pc = params['convs'][i]
        h0, h1, as16, ad16 = _tc_pre(h, pc['W'], pc['a_src'], pc['a_dst'])
        acc0, acc1, den = _sc_sparse(as16, ad16, h0, h1, srcp, dstp)
        fc = params['fcs'][i]
        nm = params['norms'][i]
        fn = params['fc_norms'][i]
        h = _tc_mid(h, acc0, acc1, den, pc['b'], nm['g'], nm['b'],
                    fc['W1'], fc['b1'], fc['W2'], fc['b2'],
                    fn['g'], fn['b'])

    return h[:N]
